# Initial kernel scaffold; baseline (speedup 1.0000x reference)
import jax
import jax.numpy as jnp
from jax.experimental import pallas as pl


def kernel(edge_index, t, msg, x_src, x_dst, W_src, b_src, W_dst, b_dst, W_self, W_msg):
    raise NotImplementedError("write your pallas kernel here")


# speedup vs baseline: 4.3516x; 4.3516x over previous
import functools

import jax
import jax.numpy as jnp
from jax import lax
from jax.experimental import pallas as pl
from jax.experimental.pallas import tpu as pltpu
from jax.experimental.pallas import tpu_sc as plsc

N_NODES = 10000
E_TOTAL = 320000
DIM = 128

NC = 2
NS = 16
NW = NC * NS
E_PER_W = E_TOTAL // NW
CHUNK = 128
N_FULL = E_PER_W // CHUNK
TAIL = E_PER_W - N_FULL * CHUNK
ROWS_PER_TILE = 640
ACC_ROWS = NS * ROWS_PER_TILE


def _mm_body(xs_ref, xd_ref, ws_ref, wd_ref, b_ref, wself_ref, wmsg_ref,
             hs_ref, hw_ref):
    hp = (jnp.dot(xs_ref[...], ws_ref[...], preferred_element_type=jnp.float32)
          + jnp.dot(xd_ref[...], wd_ref[...], preferred_element_type=jnp.float32)
          + b_ref[...])
    hs_ref[...] = jnp.dot(hp, wself_ref[...], preferred_element_type=jnp.float32)
    hw_ref[...] = jnp.dot(hp, wmsg_ref[...], preferred_element_type=jnp.float32)


def _relu_body(hs_ref, s0_ref, s1_ref, h_ref):
    h_ref[...] = jnp.maximum(hs_ref[...] + s0_ref[...] + s1_ref[...], 0.0)


def _seg_body(hw_hbm, src_hbm, dst_hbm, zeros_hbm, s_hbm,
              sidx, didx, rows, sidx_t, didx_t, rows_t, acc, sem):
    c = lax.axis_index("c")
    s = lax.axis_index("s")
    wid = s * NC + c
    stripe = pl.multiple_of(s * ROWS_PER_TILE, 8)
    pltpu.sync_copy(zeros_hbm, acc.at[pl.ds(stripe, ROWS_PER_TILE)])
    plsc.subcore_barrier()
    base = wid * E_PER_W

    def chunk(i, carry):
        off = pl.multiple_of(base + i * CHUNK, 8)
        pltpu.sync_copy(src_hbm.at[pl.ds(off, CHUNK)], sidx)
        pltpu.sync_copy(dst_hbm.at[pl.ds(off, CHUNK)], didx)
        pltpu.async_copy(hw_hbm.at[sidx], rows, sem).wait()
        pltpu.sync_copy(rows, acc.at[didx], add=True)
        return carry

    lax.fori_loop(0, N_FULL, chunk, 0)
    off = pl.multiple_of(base + N_FULL * CHUNK, 8)
    pltpu.sync_copy(src_hbm.at[pl.ds(off, TAIL)], sidx_t)
    pltpu.sync_copy(dst_hbm.at[pl.ds(off, TAIL)], didx_t)
    pltpu.async_copy(hw_hbm.at[sidx_t], rows_t, sem).wait()
    pltpu.sync_copy(rows_t, acc.at[didx_t], add=True)
    plsc.subcore_barrier()
    out_off = pl.multiple_of(c * ACC_ROWS + s * ROWS_PER_TILE, 8)
    pltpu.sync_copy(acc.at[pl.ds(stripe, ROWS_PER_TILE)],
                    s_hbm.at[pl.ds(out_off, ROWS_PER_TILE)])


def _gather_body(h_hbm, src_hbm, dst_hbm, hsrc_hbm, hdst_hbm,
                 sidx, didx, srows, drows, sidx_t, didx_t, srows_t, drows_t,
                 sem):
    c = lax.axis_index("c")
    s = lax.axis_index("s")
    wid = s * NC + c
    base = wid * E_PER_W

    def chunk(i, carry):
        off = pl.multiple_of(base + i * CHUNK, 8)
        pltpu.sync_copy(src_hbm.at[pl.ds(off, CHUNK)], sidx)
        pltpu.sync_copy(dst_hbm.at[pl.ds(off, CHUNK)], didx)
        pltpu.async_copy(h_hbm.at[sidx], srows, sem).wait()
        pltpu.async_copy(h_hbm.at[didx], drows, sem).wait()
        pltpu.sync_copy(srows, hsrc_hbm.at[pl.ds(off, CHUNK)])
        pltpu.sync_copy(drows, hdst_hbm.at[pl.ds(off, CHUNK)])
        return carry

    lax.fori_loop(0, N_FULL, chunk, 0)
    off = pl.multiple_of(base + N_FULL * CHUNK, 8)
    pltpu.sync_copy(src_hbm.at[pl.ds(off, TAIL)], sidx_t)
    pltpu.sync_copy(dst_hbm.at[pl.ds(off, TAIL)], didx_t)
    pltpu.async_copy(h_hbm.at[sidx_t], srows_t, sem).wait()
    pltpu.async_copy(h_hbm.at[didx_t], drows_t, sem).wait()
    pltpu.sync_copy(srows_t, hsrc_hbm.at[pl.ds(off, TAIL)])
    pltpu.sync_copy(drows_t, hdst_hbm.at[pl.ds(off, TAIL)])


@functools.lru_cache(maxsize=1)
def _sc_kernels():
    mesh = plsc.VectorSubcoreMesh(
        core_axis_name="c", subcore_axis_name="s",
        num_cores=NC, num_subcores=NS)
    seg = functools.partial(
        pl.kernel,
        out_type=jax.ShapeDtypeStruct((NC * ACC_ROWS, DIM), jnp.float32),
        mesh=mesh,
        scratch_types=[
            pltpu.VMEM((CHUNK,), jnp.int32),
            pltpu.VMEM((CHUNK,), jnp.int32),
            pltpu.VMEM((CHUNK, DIM), jnp.float32),
            pltpu.VMEM((TAIL,), jnp.int32),
            pltpu.VMEM((TAIL,), jnp.int32),
            pltpu.VMEM((TAIL, DIM), jnp.float32),
            pltpu.VMEM_SHARED((ACC_ROWS, DIM), jnp.float32),
            pltpu.SemaphoreType.DMA,
        ],
    )(_seg_body)
    gather = functools.partial(
        pl.kernel,
        out_type=[
            jax.ShapeDtypeStruct((E_TOTAL, DIM), jnp.float32),
            jax.ShapeDtypeStruct((E_TOTAL, DIM), jnp.float32),
        ],
        mesh=mesh,
        scratch_types=[
            pltpu.VMEM((CHUNK,), jnp.int32),
            pltpu.VMEM((CHUNK,), jnp.int32),
            pltpu.VMEM((CHUNK, DIM), jnp.float32),
            pltpu.VMEM((CHUNK, DIM), jnp.float32),
            pltpu.VMEM((TAIL,), jnp.int32),
            pltpu.VMEM((TAIL,), jnp.int32),
            pltpu.VMEM((TAIL, DIM), jnp.float32),
            pltpu.VMEM((TAIL, DIM), jnp.float32),
            pltpu.SemaphoreType.DMA,
        ],
    )(_gather_body)
    return seg, gather

_ROW_BLK = 1000
_N_BLK = N_NODES // _ROW_BLK

_mm_call = pl.pallas_call(
    _mm_body,
    grid=(_N_BLK,),
    in_specs=[
        pl.BlockSpec((_ROW_BLK, DIM), lambda i: (i, 0)),
        pl.BlockSpec((_ROW_BLK, DIM), lambda i: (i, 0)),
        pl.BlockSpec((DIM, DIM), lambda i: (0, 0)),
        pl.BlockSpec((DIM, DIM), lambda i: (0, 0)),
        pl.BlockSpec((1, DIM), lambda i: (0, 0)),
        pl.BlockSpec((DIM, DIM), lambda i: (0, 0)),
        pl.BlockSpec((DIM, DIM), lambda i: (0, 0)),
    ],
    out_specs=[
        pl.BlockSpec((_ROW_BLK, DIM), lambda i: (i, 0)),
        pl.BlockSpec((_ROW_BLK, DIM), lambda i: (i, 0)),
    ],
    out_shape=[
        jax.ShapeDtypeStruct((N_NODES, DIM), jnp.float32),
        jax.ShapeDtypeStruct((N_NODES, DIM), jnp.float32),
    ],
)

_relu_call = pl.pallas_call(
    _relu_body,
    grid=(_N_BLK,),
    in_specs=[
        pl.BlockSpec((_ROW_BLK, DIM), lambda i: (i, 0)),
        pl.BlockSpec((_ROW_BLK, DIM), lambda i: (i, 0)),
        pl.BlockSpec((_ROW_BLK, DIM), lambda i: (i, 0)),
    ],
    out_specs=pl.BlockSpec((_ROW_BLK, DIM), lambda i: (i, 0)),
    out_shape=jax.ShapeDtypeStruct((N_NODES, DIM), jnp.float32),
)


def kernel(edge_index, t, msg, x_src, x_dst, W_src, b_src, W_dst, b_dst,
           W_self, W_msg):
    del t, msg
    src = edge_index[0]
    dst = edge_index[1]
    bias = (b_src + b_dst).reshape(1, DIM)
    seg_kernel, gather_kernel = _sc_kernels()
    hs, hw = _mm_call(x_src, x_dst, W_src, W_dst, bias, W_self, W_msg)
    zeros = jnp.zeros((ROWS_PER_TILE, DIM), jnp.float32)
    s_partials = seg_kernel(hw, src, dst, zeros)
    h = _relu_call(hs, s_partials[:N_NODES],
                   s_partials[ACC_ROWS:ACC_ROWS + N_NODES])
    h_src, h_dst = gather_kernel(h, src, dst)
    return (h_src, h_dst)


# speedup vs baseline: 6.8040x; 1.5636x over previous
import functools

import jax
import jax.numpy as jnp
from jax import lax
from jax.experimental import pallas as pl
from jax.experimental.pallas import tpu as pltpu
from jax.experimental.pallas import tpu_sc as plsc

N_NODES = 10000
E_TOTAL = 320000
DIM = 128
HALF = DIM // 2

NC = 2
NS = 16
NW = NC * NS
CHUNK = 80
E_PER_W = E_TOTAL // NW
N_CHUNK_G = E_PER_W // CHUNK
N_PAIR_G = N_CHUNK_G // 2
E_PER_S = E_TOTAL // NS
N_CHUNK_S = E_PER_S // CHUNK
N_PAIR_S = N_CHUNK_S // 2
ROWS_PER_TILE = 640
ACC_ROWS = NS * ROWS_PER_TILE


def _mm_body(xs_ref, xd_ref, ws_ref, wd_ref, b_ref, wself_ref, wmsg_ref,
             hs_ref, hw0_ref, hw1_ref):
    hp = (jnp.dot(xs_ref[...], ws_ref[...], preferred_element_type=jnp.float32)
          + jnp.dot(xd_ref[...], wd_ref[...], preferred_element_type=jnp.float32)
          + b_ref[...])
    hs_ref[...] = jnp.dot(hp, wself_ref[...], preferred_element_type=jnp.float32)
    hw = jnp.dot(hp, wmsg_ref[...], preferred_element_type=jnp.float32)
    hw0_ref[...] = hw[:, :HALF]
    hw1_ref[...] = hw[:, HALF:]


def _relu_body(hs_ref, s0_ref, s1_ref, h_ref):
    s = jnp.concatenate([s0_ref[...], s1_ref[...]], axis=1)
    h_ref[...] = jnp.maximum(hs_ref[...] + s, 0.0)


def _seg_body(hw0_hbm, hw1_hbm, src_hbm, dst_hbm, zeros_hbm, s_hbm,
              sidx, didx, rows_a, rows_b, acc, g_a, g_b, sc_a, sc_b):
    c = lax.axis_index("c")
    s = lax.axis_index("s")
    stripe = pl.multiple_of(s * ROWS_PER_TILE, 8)
    pltpu.sync_copy(zeros_hbm, acc.at[pl.ds(stripe, ROWS_PER_TILE)])
    pltpu.sync_copy(src_hbm.at[s], sidx)
    pltpu.sync_copy(dst_hbm.at[s], didx)
    plsc.subcore_barrier()

    def accumulate(hw_hbm):
        pltpu.async_copy(hw_hbm.at[sidx.at[0]], rows_a, g_a)

        def pair(i, carry):
            a, b = 2 * i, 2 * i + 1

            @pl.when(i >= 1)
            def _():
                pltpu.make_async_copy(rows_b, acc.at[didx.at[0]], sc_b).wait()

            gb = pltpu.async_copy(hw_hbm.at[sidx.at[b]], rows_b, g_b)
            pltpu.make_async_copy(hw_hbm.at[sidx.at[a]], rows_a, g_a).wait()
            sca = pltpu.async_copy(rows_a, acc.at[didx.at[a]], sc_a, add=True)
            sca.wait()

            @pl.when(i < N_PAIR_S - 1)
            def _():
                pltpu.async_copy(hw_hbm.at[sidx.at[a + 2]], rows_a, g_a)

            gb.wait()
            pltpu.async_copy(rows_b, acc.at[didx.at[b]], sc_b, add=True)
            return carry

        lax.fori_loop(0, N_PAIR_S, pair, 0)
        pltpu.make_async_copy(rows_b, acc.at[didx.at[0]], sc_b).wait()

    @pl.when(c == 0)
    def _():
        accumulate(hw0_hbm)

    @pl.when(c == 1)
    def _():
        accumulate(hw1_hbm)

    plsc.subcore_barrier()
    out_off = pl.multiple_of(c * ACC_ROWS + s * ROWS_PER_TILE, 8)
    pltpu.sync_copy(acc.at[pl.ds(stripe, ROWS_PER_TILE)],
                    s_hbm.at[pl.ds(out_off, ROWS_PER_TILE)])


def _gather_body(h_hbm, src_hbm, dst_hbm, hsrc_hbm, hdst_hbm,
                 sidx, didx, srows_a, srows_b, drows_a, drows_b,
                 g_sa, g_sb, g_da, g_db, st_sa, st_sb, st_da, st_db):
    c = lax.axis_index("c")
    s = lax.axis_index("s")
    wid = s * NC + c
    base = wid * E_PER_W
    pltpu.sync_copy(src_hbm.at[wid], sidx)
    pltpu.sync_copy(dst_hbm.at[wid], didx)
    pltpu.async_copy(h_hbm.at[sidx.at[0]], srows_a, g_sa)
    pltpu.async_copy(h_hbm.at[didx.at[0]], drows_a, g_da)

    def pair(i, carry):
        a, b = 2 * i, 2 * i + 1
        off_a = pl.multiple_of(base + a * CHUNK, 8)
        off_b = pl.multiple_of(base + b * CHUNK, 8)

        @pl.when(i >= 1)
        def _():
            pltpu.make_async_copy(srows_b, hsrc_hbm.at[pl.ds(off_b, CHUNK)],
                                  st_sb).wait()
            pltpu.make_async_copy(drows_b, hdst_hbm.at[pl.ds(off_b, CHUNK)],
                                  st_db).wait()

        gsb = pltpu.async_copy(h_hbm.at[sidx.at[b]], srows_b, g_sb)
        gdb = pltpu.async_copy(h_hbm.at[didx.at[b]], drows_b, g_db)

        pltpu.make_async_copy(h_hbm.at[sidx.at[a]], srows_a, g_sa).wait()
        ssa = pltpu.async_copy(srows_a, hsrc_hbm.at[pl.ds(off_a, CHUNK)], st_sa)
        pltpu.make_async_copy(h_hbm.at[didx.at[a]], drows_a, g_da).wait()
        sda = pltpu.async_copy(drows_a, hdst_hbm.at[pl.ds(off_a, CHUNK)], st_da)

        ssa.wait()
        pltpu.async_copy(h_hbm.at[sidx.at[a + 2]], srows_a, g_sa)
        sda.wait()
        pltpu.async_copy(h_hbm.at[didx.at[a + 2]], drows_a, g_da)

        gsb.wait()
        pltpu.async_copy(srows_b, hsrc_hbm.at[pl.ds(off_b, CHUNK)], st_sb)
        gdb.wait()
        pltpu.async_copy(drows_b, hdst_hbm.at[pl.ds(off_b, CHUNK)], st_db)
        return carry

    lax.fori_loop(0, N_PAIR_G, pair, 0)
    last = N_CHUNK_G - 1
    off_l = pl.multiple_of(base + last * CHUNK, 8)
    off_b = pl.multiple_of(base + (last - 1) * CHUNK, 8)
    pltpu.make_async_copy(h_hbm.at[sidx.at[0]], srows_a, g_sa).wait()
    pltpu.async_copy(srows_a, hsrc_hbm.at[pl.ds(off_l, CHUNK)], st_sa).wait()
    pltpu.make_async_copy(h_hbm.at[didx.at[0]], drows_a, g_da).wait()
    pltpu.async_copy(drows_a, hdst_hbm.at[pl.ds(off_l, CHUNK)], st_da).wait()
    pltpu.make_async_copy(srows_b, hsrc_hbm.at[pl.ds(off_b, CHUNK)], st_sb).wait()
    pltpu.make_async_copy(drows_b, hdst_hbm.at[pl.ds(off_b, CHUNK)], st_db).wait()


@functools.lru_cache(maxsize=1)
def _sc_kernels():
    mesh = plsc.VectorSubcoreMesh(
        core_axis_name="c", subcore_axis_name="s",
        num_cores=NC, num_subcores=NS)
    seg = functools.partial(
        pl.kernel,
        out_type=jax.ShapeDtypeStruct((NC * ACC_ROWS, HALF), jnp.float32),
        mesh=mesh,
        compiler_params=pltpu.CompilerParams(use_tc_tiling_on_sc=False),
        scratch_types=[
            pltpu.VMEM((N_CHUNK_S, CHUNK), jnp.int32),
            pltpu.VMEM((N_CHUNK_S, CHUNK), jnp.int32),
            pltpu.VMEM((CHUNK, HALF), jnp.float32),
            pltpu.VMEM((CHUNK, HALF), jnp.float32),
            pltpu.VMEM_SHARED((ACC_ROWS, HALF), jnp.float32),
            pltpu.SemaphoreType.DMA,
            pltpu.SemaphoreType.DMA,
            pltpu.SemaphoreType.DMA,
            pltpu.SemaphoreType.DMA,
        ],
    )(_seg_body)
    gather = functools.partial(
        pl.kernel,
        out_type=[
            jax.ShapeDtypeStruct((E_TOTAL, DIM), jnp.float32),
            jax.ShapeDtypeStruct((E_TOTAL, DIM), jnp.float32),
        ],
        mesh=mesh,
        scratch_types=[
            pltpu.VMEM((N_CHUNK_G, CHUNK), jnp.int32),
            pltpu.VMEM((N_CHUNK_G, CHUNK), jnp.int32),
            pltpu.VMEM((CHUNK, DIM), jnp.float32),
            pltpu.VMEM((CHUNK, DIM), jnp.float32),
            pltpu.VMEM((CHUNK, DIM), jnp.float32),
            pltpu.VMEM((CHUNK, DIM), jnp.float32),
            pltpu.SemaphoreType.DMA,
            pltpu.SemaphoreType.DMA,
            pltpu.SemaphoreType.DMA,
            pltpu.SemaphoreType.DMA,
            pltpu.SemaphoreType.DMA,
            pltpu.SemaphoreType.DMA,
            pltpu.SemaphoreType.DMA,
            pltpu.SemaphoreType.DMA,
        ],
    )(_gather_body)
    return seg, gather


_ROW_BLK = 1000
_N_BLK = N_NODES // _ROW_BLK

_mm_call = pl.pallas_call(
    _mm_body,
    grid=(_N_BLK,),
    in_specs=[
        pl.BlockSpec((_ROW_BLK, DIM), lambda i: (i, 0)),
        pl.BlockSpec((_ROW_BLK, DIM), lambda i: (i, 0)),
        pl.BlockSpec((DIM, DIM), lambda i: (0, 0)),
        pl.BlockSpec((DIM, DIM), lambda i: (0, 0)),
        pl.BlockSpec((1, DIM), lambda i: (0, 0)),
        pl.BlockSpec((DIM, DIM), lambda i: (0, 0)),
        pl.BlockSpec((DIM, DIM), lambda i: (0, 0)),
    ],
    out_specs=[
        pl.BlockSpec((_ROW_BLK, DIM), lambda i: (i, 0)),
        pl.BlockSpec((_ROW_BLK, HALF), lambda i: (i, 0)),
        pl.BlockSpec((_ROW_BLK, HALF), lambda i: (i, 0)),
    ],
    out_shape=[
        jax.ShapeDtypeStruct((N_NODES, DIM), jnp.float32),
        jax.ShapeDtypeStruct((N_NODES, HALF), jnp.float32),
        jax.ShapeDtypeStruct((N_NODES, HALF), jnp.float32),
    ],
)

_relu_call = pl.pallas_call(
    _relu_body,
    grid=(_N_BLK,),
    in_specs=[
        pl.BlockSpec((_ROW_BLK, DIM), lambda i: (i, 0)),
        pl.BlockSpec((_ROW_BLK, HALF), lambda i: (i, 0)),
        pl.BlockSpec((_ROW_BLK, HALF), lambda i: (i, 0)),
    ],
    out_specs=pl.BlockSpec((_ROW_BLK, DIM), lambda i: (i, 0)),
    out_shape=jax.ShapeDtypeStruct((N_NODES, DIM), jnp.float32),
)


def kernel(edge_index, t, msg, x_src, x_dst, W_src, b_src, W_dst, b_dst,
           W_self, W_msg):
    del t, msg
    src3 = edge_index[0].reshape(NW, N_CHUNK_G, CHUNK)
    dst3 = edge_index[1].reshape(NW, N_CHUNK_G, CHUNK)
    src3s = edge_index[0].reshape(NS, N_CHUNK_S, CHUNK)
    dst3s = edge_index[1].reshape(NS, N_CHUNK_S, CHUNK)
    bias = (b_src + b_dst).reshape(1, DIM)
    seg_kernel, gather_kernel = _sc_kernels()
    hs, hw0, hw1 = _mm_call(x_src, x_dst, W_src, W_dst, bias, W_self, W_msg)
    zeros = jnp.zeros((ROWS_PER_TILE, HALF), jnp.float32)
    s_halves = seg_kernel(hw0, hw1, src3s, dst3s, zeros)
    h = _relu_call(hs, s_halves[:N_NODES],
                   s_halves[ACC_ROWS:ACC_ROWS + N_NODES])
    h_src, h_dst = gather_kernel(h, src3, dst3)
    return (h_src, h_dst)


# speedup vs baseline: 6.9017x; 1.0144x over previous
import functools

import jax
import jax.numpy as jnp
from jax import lax
from jax.experimental import pallas as pl
from jax.experimental.pallas import tpu as pltpu
from jax.experimental.pallas import tpu_sc as plsc

N_NODES = 10000
E_TOTAL = 320000
DIM = 128
HALF = DIM // 2

NC = 2
NS = 16
NW = NC * NS
CHUNK = 80
E_PER_W = E_TOTAL // NW
N_CHUNK_G = E_PER_W // CHUNK
N_PAIR_G = N_CHUNK_G // 2
E_PER_S = E_TOTAL // NS
N_CHUNK_S = E_PER_S // CHUNK
N_PAIR_S = N_CHUNK_S // 2
ROWS_PER_TILE = N_NODES // NS
ACC_ROWS = N_NODES


def _mm_body(xs_ref, xd_ref, ws_ref, wd_ref, b_ref, wself_ref, wmsg_ref,
             hs_ref, hwt_ref):
    hp = (jnp.dot(xs_ref[...], ws_ref[...], preferred_element_type=jnp.float32)
          + jnp.dot(xd_ref[...], wd_ref[...], preferred_element_type=jnp.float32)
          + b_ref[...])
    hs_ref[...] = jnp.dot(hp, wself_ref[...], preferred_element_type=jnp.float32)
    hw = jnp.dot(hp, wmsg_ref[...], preferred_element_type=jnp.float32)
    hwt_ref[...] = jnp.stack([hw[:, :HALF], hw[:, HALF:]], axis=0)


def _relu_body(hs_ref, s0_ref, s1_ref, h_ref):
    s = jnp.concatenate([s0_ref[...], s1_ref[...]], axis=1)
    h_ref[...] = jnp.maximum(hs_ref[...] + s, 0.0)


def _seg_body(hwt_hbm, src_hbm, dst_hbm, zeros_hbm, s_hbm,
              sidx, didx, rows_a, rows_b, table, acc, g_a, g_b, sc_a, sc_b):
    c = lax.axis_index("c")
    s = lax.axis_index("s")
    stripe = s * ROWS_PER_TILE
    pltpu.sync_copy(zeros_hbm, acc.at[pl.ds(stripe, ROWS_PER_TILE)])
    pltpu.sync_copy(hwt_hbm.at[c].at[pl.ds(stripe, ROWS_PER_TILE)],
                    table.at[pl.ds(stripe, ROWS_PER_TILE)])
    pltpu.sync_copy(src_hbm.at[s], sidx)
    pltpu.sync_copy(dst_hbm.at[s], didx)
    plsc.subcore_barrier()

    pltpu.async_copy(table.at[sidx.at[0]], rows_a, g_a)

    def pair(i, carry):
        a, b = 2 * i, 2 * i + 1

        @pl.when(i >= 1)
        def _():
            pltpu.make_async_copy(rows_b, acc.at[didx.at[0]], sc_b).wait()

        gb = pltpu.async_copy(table.at[sidx.at[b]], rows_b, g_b)
        pltpu.make_async_copy(table.at[sidx.at[a]], rows_a, g_a).wait()
        sca = pltpu.async_copy(rows_a, acc.at[didx.at[a]], sc_a, add=True)
        sca.wait()

        @pl.when(i < N_PAIR_S - 1)
        def _():
            pltpu.async_copy(table.at[sidx.at[a + 2]], rows_a, g_a)

        gb.wait()
        pltpu.async_copy(rows_b, acc.at[didx.at[b]], sc_b, add=True)
        return carry

    lax.fori_loop(0, N_PAIR_S, pair, 0)
    pltpu.make_async_copy(rows_b, acc.at[didx.at[0]], sc_b).wait()

    plsc.subcore_barrier()
    out_off = c * ACC_ROWS + s * ROWS_PER_TILE
    pltpu.sync_copy(acc.at[pl.ds(stripe, ROWS_PER_TILE)],
                    s_hbm.at[pl.ds(out_off, ROWS_PER_TILE)])


def _gather_body(h_hbm, src_hbm, dst_hbm, hsrc_hbm, hdst_hbm,
                 sidx, didx, srows_a, srows_b, drows_a, drows_b,
                 g_sa, g_sb, g_da, g_db, st_sa, st_sb, st_da, st_db):
    c = lax.axis_index("c")
    s = lax.axis_index("s")
    wid = s * NC + c
    base = wid * E_PER_W
    pltpu.sync_copy(src_hbm.at[wid], sidx)
    pltpu.sync_copy(dst_hbm.at[wid], didx)
    pltpu.async_copy(h_hbm.at[sidx.at[0]], srows_a, g_sa)
    pltpu.async_copy(h_hbm.at[didx.at[0]], drows_a, g_da)

    def pair(i, carry):
        a, b = 2 * i, 2 * i + 1
        off_a = pl.multiple_of(base + a * CHUNK, 8)
        off_b = pl.multiple_of(base + b * CHUNK, 8)

        @pl.when(i >= 1)
        def _():
            pltpu.make_async_copy(srows_b, hsrc_hbm.at[pl.ds(off_b, CHUNK)],
                                  st_sb).wait()
            pltpu.make_async_copy(drows_b, hdst_hbm.at[pl.ds(off_b, CHUNK)],
                                  st_db).wait()

        gsb = pltpu.async_copy(h_hbm.at[sidx.at[b]], srows_b, g_sb)
        gdb = pltpu.async_copy(h_hbm.at[didx.at[b]], drows_b, g_db)

        pltpu.make_async_copy(h_hbm.at[sidx.at[a]], srows_a, g_sa).wait()
        ssa = pltpu.async_copy(srows_a, hsrc_hbm.at[pl.ds(off_a, CHUNK)], st_sa)
        pltpu.make_async_copy(h_hbm.at[didx.at[a]], drows_a, g_da).wait()
        sda = pltpu.async_copy(drows_a, hdst_hbm.at[pl.ds(off_a, CHUNK)], st_da)

        ssa.wait()
        pltpu.async_copy(h_hbm.at[sidx.at[a + 2]], srows_a, g_sa)
        sda.wait()
        pltpu.async_copy(h_hbm.at[didx.at[a + 2]], drows_a, g_da)

        gsb.wait()
        pltpu.async_copy(srows_b, hsrc_hbm.at[pl.ds(off_b, CHUNK)], st_sb)
        gdb.wait()
        pltpu.async_copy(drows_b, hdst_hbm.at[pl.ds(off_b, CHUNK)], st_db)
        return carry

    lax.fori_loop(0, N_PAIR_G, pair, 0)
    last = N_CHUNK_G - 1
    off_l = pl.multiple_of(base + last * CHUNK, 8)
    off_b = pl.multiple_of(base + (last - 1) * CHUNK, 8)
    pltpu.make_async_copy(h_hbm.at[sidx.at[0]], srows_a, g_sa).wait()
    pltpu.async_copy(srows_a, hsrc_hbm.at[pl.ds(off_l, CHUNK)], st_sa).wait()
    pltpu.make_async_copy(h_hbm.at[didx.at[0]], drows_a, g_da).wait()
    pltpu.async_copy(drows_a, hdst_hbm.at[pl.ds(off_l, CHUNK)], st_da).wait()
    pltpu.make_async_copy(srows_b, hsrc_hbm.at[pl.ds(off_b, CHUNK)], st_sb).wait()
    pltpu.make_async_copy(drows_b, hdst_hbm.at[pl.ds(off_b, CHUNK)], st_db).wait()


@functools.lru_cache(maxsize=1)
def _sc_kernels():
    mesh = plsc.VectorSubcoreMesh(
        core_axis_name="c", subcore_axis_name="s",
        num_cores=NC, num_subcores=NS)
    seg = functools.partial(
        pl.kernel,
        out_type=jax.ShapeDtypeStruct((NC * ACC_ROWS, HALF), jnp.float32),
        mesh=mesh,
        compiler_params=pltpu.CompilerParams(use_tc_tiling_on_sc=False),
        scratch_types=[
            pltpu.VMEM((N_CHUNK_S, CHUNK), jnp.int32),
            pltpu.VMEM((N_CHUNK_S, CHUNK), jnp.int32),
            pltpu.VMEM((CHUNK, HALF), jnp.float32),
            pltpu.VMEM((CHUNK, HALF), jnp.float32),
            pltpu.VMEM_SHARED((ACC_ROWS, HALF), jnp.float32),
            pltpu.VMEM_SHARED((ACC_ROWS, HALF), jnp.float32),
            pltpu.SemaphoreType.DMA,
            pltpu.SemaphoreType.DMA,
            pltpu.SemaphoreType.DMA,
            pltpu.SemaphoreType.DMA,
        ],
    )(_seg_body)
    gather = functools.partial(
        pl.kernel,
        out_type=[
            jax.ShapeDtypeStruct((E_TOTAL, DIM), jnp.float32),
            jax.ShapeDtypeStruct((E_TOTAL, DIM), jnp.float32),
        ],
        mesh=mesh,
        scratch_types=[
            pltpu.VMEM((N_CHUNK_G, CHUNK), jnp.int32),
            pltpu.VMEM((N_CHUNK_G, CHUNK), jnp.int32),
            pltpu.VMEM((CHUNK, DIM), jnp.float32),
            pltpu.VMEM((CHUNK, DIM), jnp.float32),
            pltpu.VMEM((CHUNK, DIM), jnp.float32),
            pltpu.VMEM((CHUNK, DIM), jnp.float32),
            pltpu.SemaphoreType.DMA,
            pltpu.SemaphoreType.DMA,
            pltpu.SemaphoreType.DMA,
            pltpu.SemaphoreType.DMA,
            pltpu.SemaphoreType.DMA,
            pltpu.SemaphoreType.DMA,
            pltpu.SemaphoreType.DMA,
            pltpu.SemaphoreType.DMA,
        ],
    )(_gather_body)
    return seg, gather


_ROW_BLK = 1000
_N_BLK = N_NODES // _ROW_BLK

_mm_call = pl.pallas_call(
    _mm_body,
    grid=(_N_BLK,),
    in_specs=[
        pl.BlockSpec((_ROW_BLK, DIM), lambda i: (i, 0)),
        pl.BlockSpec((_ROW_BLK, DIM), lambda i: (i, 0)),
        pl.BlockSpec((DIM, DIM), lambda i: (0, 0)),
        pl.BlockSpec((DIM, DIM), lambda i: (0, 0)),
        pl.BlockSpec((1, DIM), lambda i: (0, 0)),
        pl.BlockSpec((DIM, DIM), lambda i: (0, 0)),
        pl.BlockSpec((DIM, DIM), lambda i: (0, 0)),
    ],
    out_specs=[
        pl.BlockSpec((_ROW_BLK, DIM), lambda i: (i, 0)),
        pl.BlockSpec((NC, _ROW_BLK, HALF), lambda i: (0, i, 0)),
    ],
    out_shape=[
        jax.ShapeDtypeStruct((N_NODES, DIM), jnp.float32),
        jax.ShapeDtypeStruct((NC, N_NODES, HALF), jnp.float32),
    ],
)

_relu_call = pl.pallas_call(
    _relu_body,
    grid=(_N_BLK,),
    in_specs=[
        pl.BlockSpec((_ROW_BLK, DIM), lambda i: (i, 0)),
        pl.BlockSpec((_ROW_BLK, HALF), lambda i: (i, 0)),
        pl.BlockSpec((_ROW_BLK, HALF), lambda i: (i, 0)),
    ],
    out_specs=pl.BlockSpec((_ROW_BLK, DIM), lambda i: (i, 0)),
    out_shape=jax.ShapeDtypeStruct((N_NODES, DIM), jnp.float32),
)


def kernel(edge_index, t, msg, x_src, x_dst, W_src, b_src, W_dst, b_dst,
           W_self, W_msg):
    del t, msg
    src3 = edge_index[0].reshape(NW, N_CHUNK_G, CHUNK)
    dst3 = edge_index[1].reshape(NW, N_CHUNK_G, CHUNK)
    src3s = edge_index[0].reshape(NS, N_CHUNK_S, CHUNK)
    dst3s = edge_index[1].reshape(NS, N_CHUNK_S, CHUNK)
    bias = (b_src + b_dst).reshape(1, DIM)
    seg_kernel, gather_kernel = _sc_kernels()
    hs, hwt = _mm_call(x_src, x_dst, W_src, W_dst, bias, W_self, W_msg)
    zeros = jnp.zeros((ROWS_PER_TILE, HALF), jnp.float32)
    s_halves = seg_kernel(hwt, src3s, dst3s, zeros)
    h = _relu_call(hs, s_halves[:N_NODES],
                   s_halves[ACC_ROWS:ACC_ROWS + N_NODES])
    h_src, h_dst = gather_kernel(h, src3, dst3)
    return (h_src, h_dst)


# speedup vs baseline: 8.6100x; 1.2475x over previous
import functools

import jax
import jax.numpy as jnp
from jax import lax
from jax.experimental import pallas as pl
from jax.experimental.pallas import tpu as pltpu
from jax.experimental.pallas import tpu_sc as plsc

N_NODES = 10000
E_TOTAL = 320000
DIM = 128
HALF = DIM // 2

NC = 2
NS = 16
NW = NC * NS
CHUNK = 80
E_PER_W = E_TOTAL // NW
N_CHUNK_G = E_PER_W // CHUNK
N_PAIR_G = N_CHUNK_G // 2
E_PER_S = E_TOTAL // NS
N_CHUNK_S = E_PER_S // CHUNK
N_PAIR_S = N_CHUNK_S // 2
ROWS_PER_TILE = N_NODES // NS
ACC_ROWS = N_NODES


def _mm_body(xs_ref, xd_ref, ws_ref, wd_ref, b_ref, wself_ref, wmsg_ref,
             hs_ref, hwt_ref):
    hp = (jnp.dot(xs_ref[...], ws_ref[...], preferred_element_type=jnp.float32)
          + jnp.dot(xd_ref[...], wd_ref[...], preferred_element_type=jnp.float32)
          + b_ref[...])
    hs_ref[...] = jnp.dot(hp, wself_ref[...], preferred_element_type=jnp.float32)
    hw = jnp.dot(hp, wmsg_ref[...], preferred_element_type=jnp.float32)
    hwt_ref[...] = jnp.stack([hw[:, :HALF], hw[:, HALF:]], axis=0)


def _relu_body(hs_ref, s0_ref, s1_ref, h_ref):
    s = jnp.concatenate([s0_ref[...], s1_ref[...]], axis=1)
    h = jnp.maximum(hs_ref[...] + s, 0.0)
    h_ref[...] = jnp.stack([h[:, :HALF], h[:, HALF:]], axis=0)


def _seg_body(hwt_hbm, src_hbm, dst_hbm, zeros_hbm, s_hbm,
              sidx, didx, rows_a, rows_b, table, acc, g_a, g_b, sc_a, sc_b):
    c = lax.axis_index("c")
    s = lax.axis_index("s")
    stripe = s * ROWS_PER_TILE
    pltpu.sync_copy(zeros_hbm, acc.at[pl.ds(stripe, ROWS_PER_TILE)])
    pltpu.sync_copy(hwt_hbm.at[c].at[pl.ds(stripe, ROWS_PER_TILE)],
                    table.at[pl.ds(stripe, ROWS_PER_TILE)])
    pltpu.sync_copy(src_hbm.at[s], sidx)
    pltpu.sync_copy(dst_hbm.at[s], didx)
    plsc.subcore_barrier()

    pltpu.async_copy(table.at[sidx.at[0]], rows_a, g_a)

    def pair(i, carry):
        a, b = 2 * i, 2 * i + 1

        @pl.when(i >= 1)
        def _():
            pltpu.make_async_copy(rows_b, acc.at[didx.at[0]], sc_b).wait()

        gb = pltpu.async_copy(table.at[sidx.at[b]], rows_b, g_b)
        pltpu.make_async_copy(table.at[sidx.at[a]], rows_a, g_a).wait()
        sca = pltpu.async_copy(rows_a, acc.at[didx.at[a]], sc_a, add=True)
        sca.wait()

        @pl.when(i < N_PAIR_S - 1)
        def _():
            pltpu.async_copy(table.at[sidx.at[a + 2]], rows_a, g_a)

        gb.wait()
        pltpu.async_copy(rows_b, acc.at[didx.at[b]], sc_b, add=True)
        return carry

    lax.fori_loop(0, N_PAIR_S, pair, 0)
    pltpu.make_async_copy(rows_b, acc.at[didx.at[0]], sc_b).wait()

    plsc.subcore_barrier()
    out_off = c * ACC_ROWS + s * ROWS_PER_TILE
    pltpu.sync_copy(acc.at[pl.ds(stripe, ROWS_PER_TILE)],
                    s_hbm.at[pl.ds(out_off, ROWS_PER_TILE)])


def _gather_body(ht_hbm, src_hbm, dst_hbm, hsrc_hbm, hdst_hbm,
                 sidx, didx, srows_a, srows_b, drows_a, drows_b, table,
                 g_sa, g_sb, g_da, g_db, st_sa, st_sb, st_da, st_db):
    c = lax.axis_index("c")
    s = lax.axis_index("s")
    base = s * E_PER_S
    col = c * HALF
    stripe = s * ROWS_PER_TILE
    pltpu.sync_copy(ht_hbm.at[c].at[pl.ds(stripe, ROWS_PER_TILE)],
                    table.at[pl.ds(stripe, ROWS_PER_TILE)])
    pltpu.sync_copy(src_hbm.at[s], sidx)
    pltpu.sync_copy(dst_hbm.at[s], didx)
    plsc.subcore_barrier()
    pltpu.async_copy(table.at[sidx.at[0]], srows_a, g_sa)
    pltpu.async_copy(table.at[didx.at[0]], drows_a, g_da)

    def pair(i, carry):
        a, b = 2 * i, 2 * i + 1
        off_a = base + a * CHUNK
        off_b = base + b * CHUNK

        @pl.when(i >= 1)
        def _():
            pltpu.make_async_copy(
                srows_b, hsrc_hbm.at[pl.ds(off_b, CHUNK), pl.ds(col, HALF)],
                st_sb).wait()
            pltpu.make_async_copy(
                drows_b, hdst_hbm.at[pl.ds(off_b, CHUNK), pl.ds(col, HALF)],
                st_db).wait()

        gsb = pltpu.async_copy(table.at[sidx.at[b]], srows_b, g_sb)
        gdb = pltpu.async_copy(table.at[didx.at[b]], drows_b, g_db)

        pltpu.make_async_copy(table.at[sidx.at[a]], srows_a, g_sa).wait()
        ssa = pltpu.async_copy(
            srows_a, hsrc_hbm.at[pl.ds(off_a, CHUNK), pl.ds(col, HALF)], st_sa)
        pltpu.make_async_copy(table.at[didx.at[a]], drows_a, g_da).wait()
        sda = pltpu.async_copy(
            drows_a, hdst_hbm.at[pl.ds(off_a, CHUNK), pl.ds(col, HALF)], st_da)

        ssa.wait()

        @pl.when(i < N_PAIR_S - 1)
        def _():
            pltpu.async_copy(table.at[sidx.at[a + 2]], srows_a, g_sa)

        sda.wait()

        @pl.when(i < N_PAIR_S - 1)
        def _():
            pltpu.async_copy(table.at[didx.at[a + 2]], drows_a, g_da)

        gsb.wait()
        pltpu.async_copy(srows_b, hsrc_hbm.at[pl.ds(off_b, CHUNK),
                                              pl.ds(col, HALF)], st_sb)
        gdb.wait()
        pltpu.async_copy(drows_b, hdst_hbm.at[pl.ds(off_b, CHUNK),
                                              pl.ds(col, HALF)], st_db)
        return carry

    lax.fori_loop(0, N_PAIR_S, pair, 0)
    off_b = base + (N_CHUNK_S - 1) * CHUNK
    pltpu.make_async_copy(
        srows_b, hsrc_hbm.at[pl.ds(off_b, CHUNK), pl.ds(col, HALF)],
        st_sb).wait()
    pltpu.make_async_copy(
        drows_b, hdst_hbm.at[pl.ds(off_b, CHUNK), pl.ds(col, HALF)],
        st_db).wait()


@functools.lru_cache(maxsize=1)
def _sc_kernels():
    mesh = plsc.VectorSubcoreMesh(
        core_axis_name="c", subcore_axis_name="s",
        num_cores=NC, num_subcores=NS)
    seg = functools.partial(
        pl.kernel,
        out_type=jax.ShapeDtypeStruct((NC * ACC_ROWS, HALF), jnp.float32),
        mesh=mesh,
        compiler_params=pltpu.CompilerParams(use_tc_tiling_on_sc=False),
        scratch_types=[
            pltpu.VMEM((N_CHUNK_S, CHUNK), jnp.int32),
            pltpu.VMEM((N_CHUNK_S, CHUNK), jnp.int32),
            pltpu.VMEM((CHUNK, HALF), jnp.float32),
            pltpu.VMEM((CHUNK, HALF), jnp.float32),
            pltpu.VMEM_SHARED((ACC_ROWS, HALF), jnp.float32),
            pltpu.VMEM_SHARED((ACC_ROWS, HALF), jnp.float32),
            pltpu.SemaphoreType.DMA,
            pltpu.SemaphoreType.DMA,
            pltpu.SemaphoreType.DMA,
            pltpu.SemaphoreType.DMA,
        ],
    )(_seg_body)
    gather = functools.partial(
        pl.kernel,
        out_type=[
            jax.ShapeDtypeStruct((E_TOTAL, DIM), jnp.float32),
            jax.ShapeDtypeStruct((E_TOTAL, DIM), jnp.float32),
        ],
        mesh=mesh,
        compiler_params=pltpu.CompilerParams(use_tc_tiling_on_sc=False),
        scratch_types=[
            pltpu.VMEM((N_CHUNK_S, CHUNK), jnp.int32),
            pltpu.VMEM((N_CHUNK_S, CHUNK), jnp.int32),
            pltpu.VMEM((CHUNK, HALF), jnp.float32),
            pltpu.VMEM((CHUNK, HALF), jnp.float32),
            pltpu.VMEM((CHUNK, HALF), jnp.float32),
            pltpu.VMEM((CHUNK, HALF), jnp.float32),
            pltpu.VMEM_SHARED((N_NODES, HALF), jnp.float32),
            pltpu.SemaphoreType.DMA,
            pltpu.SemaphoreType.DMA,
            pltpu.SemaphoreType.DMA,
            pltpu.SemaphoreType.DMA,
            pltpu.SemaphoreType.DMA,
            pltpu.SemaphoreType.DMA,
            pltpu.SemaphoreType.DMA,
            pltpu.SemaphoreType.DMA,
        ],
    )(_gather_body)
    return seg, gather


_ROW_BLK = 1000
_N_BLK = N_NODES // _ROW_BLK

_mm_call = pl.pallas_call(
    _mm_body,
    grid=(_N_BLK,),
    in_specs=[
        pl.BlockSpec((_ROW_BLK, DIM), lambda i: (i, 0)),
        pl.BlockSpec((_ROW_BLK, DIM), lambda i: (i, 0)),
        pl.BlockSpec((DIM, DIM), lambda i: (0, 0)),
        pl.BlockSpec((DIM, DIM), lambda i: (0, 0)),
        pl.BlockSpec((1, DIM), lambda i: (0, 0)),
        pl.BlockSpec((DIM, DIM), lambda i: (0, 0)),
        pl.BlockSpec((DIM, DIM), lambda i: (0, 0)),
    ],
    out_specs=[
        pl.BlockSpec((_ROW_BLK, DIM), lambda i: (i, 0)),
        pl.BlockSpec((NC, _ROW_BLK, HALF), lambda i: (0, i, 0)),
    ],
    out_shape=[
        jax.ShapeDtypeStruct((N_NODES, DIM), jnp.float32),
        jax.ShapeDtypeStruct((NC, N_NODES, HALF), jnp.float32),
    ],
)

_relu_call = pl.pallas_call(
    _relu_body,
    grid=(_N_BLK,),
    in_specs=[
        pl.BlockSpec((_ROW_BLK, DIM), lambda i: (i, 0)),
        pl.BlockSpec((_ROW_BLK, HALF), lambda i: (i, 0)),
        pl.BlockSpec((_ROW_BLK, HALF), lambda i: (i, 0)),
    ],
    out_specs=pl.BlockSpec((NC, _ROW_BLK, HALF), lambda i: (0, i, 0)),
    out_shape=jax.ShapeDtypeStruct((NC, N_NODES, HALF), jnp.float32),
)


def kernel(edge_index, t, msg, x_src, x_dst, W_src, b_src, W_dst, b_dst,
           W_self, W_msg):
    del t, msg
    src3s = edge_index[0].reshape(NS, N_CHUNK_S, CHUNK)
    dst3s = edge_index[1].reshape(NS, N_CHUNK_S, CHUNK)
    bias = (b_src + b_dst).reshape(1, DIM)
    seg_kernel, gather_kernel = _sc_kernels()
    hs, hwt = _mm_call(x_src, x_dst, W_src, W_dst, bias, W_self, W_msg)
    zeros = jnp.zeros((ROWS_PER_TILE, HALF), jnp.float32)
    s_halves = seg_kernel(hwt, src3s, dst3s, zeros)
    ht = _relu_call(hs, s_halves[:N_NODES],
                    s_halves[ACC_ROWS:ACC_ROWS + N_NODES])
    h_src, h_dst = gather_kernel(ht, src3s, dst3s)
    return (h_src, h_dst)
